# Initial kernel scaffold; baseline (speedup 1.0000x reference)
import jax
import jax.numpy as jnp
from jax.experimental import pallas as pl


def kernel(dense_input, sparse_input, emb_table, fm_W, fm_b):
    raise NotImplementedError("write your pallas kernel here")


# speedup vs baseline: 45.7981x; 45.7981x over previous
import functools

import jax
import jax.numpy as jnp
from jax import lax
from jax.experimental import pallas as pl
from jax.experimental.pallas import tpu as pltpu
from jax.experimental.pallas import tpu_sc as plsc

_L = 16


def _sigmoid(x):
    return 1.0 / (1.0 + jnp.exp(-x))


@functools.partial(jax.jit, static_argnums=(5, 6, 7))
def _run(table, idx, dns, wrow, brow, num_workers, num_groups, num_fields):
    vpad = table.shape[0]
    ndense = dns.shape[2]
    mesh = plsc.VectorSubcoreMesh(core_axis_name="c", subcore_axis_name="s")

    @functools.partial(
        pl.kernel,
        mesh=mesh,
        out_type=jax.ShapeDtypeStruct((num_workers, num_groups, _L), jnp.float32),
        scratch_types=[
            pltpu.VMEM((vpad,), jnp.float32),
            pltpu.VMEM((num_groups, num_fields, _L), jnp.int32),
            pltpu.VMEM((num_groups, ndense, _L), jnp.float32),
            pltpu.VMEM((ndense, _L), jnp.float32),
            pltpu.VMEM((_L,), jnp.float32),
            pltpu.VMEM((num_groups, _L), jnp.float32),
        ],
        compiler_params=pltpu.CompilerParams(
            needs_layout_passes=False, use_tc_tiling_on_sc=False
        ),
    )
    def k(table_hbm, idx_hbm, dns_hbm, w_hbm, b_hbm, out_hbm,
          table_v, idx_v, dns_v, w_v, b_v, out_v):
        wid = lax.axis_index("s") * 2 + lax.axis_index("c")
        pltpu.sync_copy(table_hbm, table_v)
        pltpu.sync_copy(idx_hbm.at[wid], idx_v)
        pltpu.sync_copy(dns_hbm.at[wid], dns_v)
        pltpu.sync_copy(w_hbm, w_v)
        pltpu.sync_copy(b_hbm, b_v)

        def body(g, _):
            acc = b_v[...]
            for j in range(ndense):
                acc = acc + dns_v[g, j] * w_v[j]
            for f in range(num_fields):
                acc = acc + plsc.load_gather(table_v, [idx_v[g, f]])
            out_v[g] = _sigmoid(acc)
            return _

        lax.fori_loop(0, num_groups, body, None)
        pltpu.sync_copy(out_v, out_hbm.at[wid])

    return k(table, idx, dns, wrow, brow)


def kernel(dense_input, sparse_input, emb_table, fm_W, fm_b):
    B, ndense = dense_input.shape
    F = sparse_input.shape[1]
    V = emb_table.shape[0]
    NW = 32
    bpw = B // NW
    G = bpw // _L

    idx = sparse_input.astype(jnp.int32).reshape(NW, G, _L, F)
    idx = jnp.swapaxes(idx, 2, 3)
    dns = dense_input.astype(jnp.float32).reshape(NW, G, _L, ndense)
    dns = jnp.swapaxes(dns, 2, 3)
    vpad = ((V + _L - 1) // _L) * _L
    table = jnp.zeros((vpad,), jnp.float32).at[:V].set(emb_table[:, 0])
    wrow = jnp.broadcast_to(fm_W.reshape(ndense, 1), (ndense, _L)).astype(jnp.float32)
    brow = jnp.broadcast_to(fm_b.reshape(1), (_L,)).astype(jnp.float32)

    out = _run(table, idx, dns, wrow, brow, NW, G, F)
    return out.reshape(B, 1)


# speedup vs baseline: 59.0594x; 1.2896x over previous
import functools

import jax
import jax.numpy as jnp
from jax import lax
from jax.experimental import pallas as pl
from jax.experimental.pallas import tpu as pltpu
from jax.experimental.pallas import tpu_sc as plsc

_L = 16


def _sigmoid(x):
    return 1.0 / (1.0 + jnp.exp(-x))


@functools.partial(jax.jit, static_argnums=(5, 6, 7, 8))
def _run(table, idx, dns, wrow, brow, num_workers, num_groups, num_fields, ndense):
    vpad = table.shape[0]
    bpw = num_groups * _L
    mesh = plsc.VectorSubcoreMesh(core_axis_name="c", subcore_axis_name="s")

    @functools.partial(
        pl.kernel,
        mesh=mesh,
        out_type=jax.ShapeDtypeStruct((num_workers, num_groups, _L), jnp.float32),
        scratch_types=[
            pltpu.VMEM((vpad,), jnp.float32),
            pltpu.VMEM((bpw * num_fields,), jnp.int32),
            pltpu.VMEM((bpw * ndense,), jnp.float32),
            pltpu.VMEM((ndense, _L), jnp.float32),
            pltpu.VMEM((_L,), jnp.float32),
            pltpu.VMEM((num_groups, _L), jnp.float32),
        ],
        compiler_params=pltpu.CompilerParams(
            needs_layout_passes=False, use_tc_tiling_on_sc=False
        ),
    )
    def k(table_hbm, idx_hbm, dns_hbm, w_hbm, b_hbm, out_hbm,
          table_v, idx_v, dns_v, w_v, b_v, out_v):
        wid = lax.axis_index("s") * 2 + lax.axis_index("c")
        pltpu.sync_copy(table_hbm, table_v)
        pltpu.sync_copy(idx_hbm.at[wid], idx_v)
        pltpu.sync_copy(dns_hbm.at[wid], dns_v)
        pltpu.sync_copy(w_hbm, w_v)
        pltpu.sync_copy(b_hbm, b_v)

        lane = lax.iota(jnp.int32, _L)
        lane_f = lane * num_fields
        lane_d = lane * ndense

        def body(g, _):
            acc = b_v[...]
            for j in range(ndense):
                pos = lane_d + (g * (_L * ndense) + j)
                acc = acc + plsc.load_gather(dns_v, [pos]) * w_v[j]
            for f in range(num_fields):
                pos = lane_f + (g * (_L * num_fields) + f)
                ii = plsc.load_gather(idx_v, [pos])
                acc = acc + plsc.load_gather(table_v, [ii])
            out_v[g] = _sigmoid(acc)
            return _

        lax.fori_loop(0, num_groups, body, None)
        pltpu.sync_copy(out_v, out_hbm.at[wid])

    return k(table, idx, dns, wrow, brow)


def kernel(dense_input, sparse_input, emb_table, fm_W, fm_b):
    B, ndense = dense_input.shape
    F = sparse_input.shape[1]
    V = emb_table.shape[0]
    NW = 32
    bpw = B // NW
    G = bpw // _L

    idx = sparse_input.astype(jnp.int32).reshape(NW, bpw * F)
    dns = dense_input.astype(jnp.float32).reshape(NW, bpw * ndense)
    vpad = ((V + _L - 1) // _L) * _L
    table = jnp.zeros((vpad,), jnp.float32).at[:V].set(emb_table[:, 0])
    wrow = jnp.broadcast_to(fm_W.reshape(ndense, 1), (ndense, _L)).astype(jnp.float32)
    brow = jnp.broadcast_to(fm_b.reshape(1), (_L,)).astype(jnp.float32)

    out = _run(table, idx, dns, wrow, brow, NW, G, F, ndense)
    return out.reshape(B, 1)


# speedup vs baseline: 66.9157x; 1.1330x over previous
import functools

import jax
import jax.numpy as jnp
from jax import lax
from jax.experimental import pallas as pl
from jax.experimental.pallas import tpu as pltpu
from jax.experimental.pallas import tpu_sc as plsc

_L = 16
_CHUNK = 256


def _sigmoid(x):
    return 1.0 / (1.0 + jnp.exp(-x))


@functools.partial(jax.jit, static_argnums=(5, 6, 7))
def _run(table, idx, dns, wflat, brow, num_workers, num_fields, ndense):
    vpad = table.shape[0]
    B = idx.shape[0]
    bpw = B // num_workers
    nchunks = bpw // _CHUNK
    ngroups = _CHUNK // _L
    mesh = plsc.VectorSubcoreMesh(core_axis_name="c", subcore_axis_name="s")

    @functools.partial(
        pl.kernel,
        mesh=mesh,
        out_type=jax.ShapeDtypeStruct((B,), jnp.float32),
        scratch_types=[
            pltpu.VMEM((vpad,), jnp.float32),
            pltpu.VMEM((_CHUNK, num_fields), jnp.int32),
            pltpu.VMEM((_CHUNK, ndense), jnp.float32),
            pltpu.VMEM((ndense * _L,), jnp.float32),
            pltpu.VMEM((_L,), jnp.float32),
            pltpu.VMEM((bpw,), jnp.float32),
        ],
        compiler_params=pltpu.CompilerParams(
            needs_layout_passes=False, use_tc_tiling_on_sc=True
        ),
    )
    def k(table_hbm, idx_hbm, dns_hbm, w_hbm, b_hbm, out_hbm,
          table_v, idx_v, dns_v, w_v, b_v, out_v):
        wid = lax.axis_index("s") * 2 + lax.axis_index("c")
        pltpu.sync_copy(table_hbm, table_v)
        pltpu.sync_copy(w_hbm, w_v)
        pltpu.sync_copy(b_hbm, b_v)

        lane = lax.iota(jnp.int32, _L)
        zero = jnp.zeros((_L,), jnp.int32)

        for c in range(nchunks):
            cbase = wid * bpw + c * _CHUNK
            pltpu.sync_copy(idx_hbm.at[pl.ds(cbase, _CHUNK)], idx_v)
            pltpu.sync_copy(dns_hbm.at[pl.ds(cbase, _CHUNK)], dns_v)

            def body(g, _, c=c):
                rows = g * _L + lane
                acc = b_v[...]
                for j in range(ndense):
                    dv = plsc.load_gather(dns_v, [rows, zero + j])
                    acc = acc + dv * w_v[pl.ds(j * _L, _L)]
                for f in range(num_fields):
                    ii = plsc.load_gather(idx_v, [rows, zero + f])
                    acc = acc + plsc.load_gather(table_v, [ii])
                out_v[pl.ds(c * _CHUNK + g * _L, _L)] = _sigmoid(acc)
                return _

            lax.fori_loop(0, ngroups, body, None)

        pltpu.sync_copy(out_v, out_hbm.at[pl.ds(wid * bpw, bpw)])

    return k(table, idx, dns, wflat, brow)


def kernel(dense_input, sparse_input, emb_table, fm_W, fm_b):
    B, ndense = dense_input.shape
    F = sparse_input.shape[1]
    V = emb_table.shape[0]
    NW = 32

    idx = sparse_input.astype(jnp.int32)
    dns = dense_input.astype(jnp.float32)
    vpad = ((V + 127) // 128) * 128
    table = jnp.zeros((vpad,), jnp.float32).at[:V].set(emb_table[:, 0])
    wflat = jnp.broadcast_to(
        fm_W.reshape(ndense, 1), (ndense, _L)
    ).astype(jnp.float32).reshape(ndense * _L)
    brow = jnp.broadcast_to(fm_b.reshape(1), (_L,)).astype(jnp.float32)

    out = _run(table, idx, dns, wflat, brow, NW, F, ndense)
    return out.reshape(B, 1)


# speedup vs baseline: 118.5086x; 1.7710x over previous
import functools

import jax
import jax.numpy as jnp
from jax import lax
from jax.experimental import pallas as pl
from jax.experimental.pallas import tpu as pltpu
from jax.experimental.pallas import tpu_sc as plsc

_L = 16


def _sigmoid(x):
    return 1.0 / (1.0 + jnp.exp(-x))


@functools.partial(jax.jit, static_argnums=(5,))
def _run(table, idx_t, dns_t, wflat, brow, num_workers):
    vpad = table.shape[0]
    num_fields, B = idx_t.shape
    ndense = dns_t.shape[0]
    bpw = B // num_workers
    ngroups = bpw // _L
    mesh = plsc.VectorSubcoreMesh(core_axis_name="c", subcore_axis_name="s")

    @functools.partial(
        pl.kernel,
        mesh=mesh,
        out_type=jax.ShapeDtypeStruct((B,), jnp.float32),
        scratch_types=[
            pltpu.VMEM((vpad,), jnp.float32),
            pltpu.VMEM((num_fields, bpw), jnp.int32),
            pltpu.VMEM((ndense, bpw), jnp.float32),
            pltpu.VMEM((ndense * _L,), jnp.float32),
            pltpu.VMEM((_L,), jnp.float32),
            pltpu.VMEM((bpw,), jnp.float32),
        ],
        compiler_params=pltpu.CompilerParams(
            needs_layout_passes=False, use_tc_tiling_on_sc=True
        ),
    )
    def k(table_hbm, idx_hbm, dns_hbm, w_hbm, b_hbm, out_hbm,
          table_v, idx_v, dns_v, w_v, b_v, out_v):
        wid = lax.axis_index("s") * 2 + lax.axis_index("c")
        base = wid * bpw
        pltpu.sync_copy(table_hbm, table_v)
        pltpu.sync_copy(idx_hbm.at[:, pl.ds(base, bpw)], idx_v)
        pltpu.sync_copy(dns_hbm.at[:, pl.ds(base, bpw)], dns_v)
        pltpu.sync_copy(w_hbm, w_v)
        pltpu.sync_copy(b_hbm, b_v)

        lane = lax.iota(jnp.int32, _L)
        zero = jnp.zeros((_L,), jnp.int32)

        def body(g, _):
            rows = g * _L + lane
            acc = b_v[...]
            for j in range(ndense):
                dv = plsc.load_gather(dns_v, [zero + j, rows])
                acc = acc + dv * w_v[pl.ds(j * _L, _L)]
            for f in range(num_fields):
                ii = plsc.load_gather(idx_v, [zero + f, rows])
                acc = acc + plsc.load_gather(table_v, [ii])
            out_v[pl.ds(g * _L, _L)] = _sigmoid(acc)
            return _

        lax.fori_loop(0, ngroups, body, None)
        pltpu.sync_copy(out_v, out_hbm.at[pl.ds(base, bpw)])

    return k(table, idx_t, dns_t, wflat, brow)


def kernel(dense_input, sparse_input, emb_table, fm_W, fm_b):
    B, ndense = dense_input.shape
    V = emb_table.shape[0]
    NW = 32

    idx_t = sparse_input.astype(jnp.int32).T
    dns_t = dense_input.astype(jnp.float32).T
    vpad = ((V + 127) // 128) * 128
    table = jnp.zeros((vpad,), jnp.float32).at[:V].set(emb_table[:, 0])
    wflat = jnp.broadcast_to(
        fm_W.reshape(ndense, 1), (ndense, _L)
    ).astype(jnp.float32).reshape(ndense * _L)
    brow = jnp.broadcast_to(fm_b.reshape(1), (_L,)).astype(jnp.float32)

    out = _run(table, idx_t, dns_t, wflat, brow, NW)
    return out.reshape(B, 1)


# speedup vs baseline: 134.7745x; 1.1373x over previous
import functools

import jax
import jax.numpy as jnp
from jax import lax
from jax.experimental import pallas as pl
from jax.experimental.pallas import tpu as pltpu
from jax.experimental.pallas import tpu_sc as plsc

_L = 16


def _sigmoid(x):
    return 1.0 / (1.0 + jnp.exp(-x))


@functools.partial(jax.jit, static_argnums=(4,))
def _run(table, idx_t, dns_t, wb, num_workers):
    vpad = table.shape[0]
    num_fields, B = idx_t.shape
    ndense = dns_t.shape[0]
    bpw = B // num_workers
    ngroups = bpw // _L
    mesh = plsc.VectorSubcoreMesh(core_axis_name="c", subcore_axis_name="s")

    @functools.partial(
        pl.kernel,
        mesh=mesh,
        out_type=jax.ShapeDtypeStruct((B,), jnp.float32),
        scratch_types=[
            pltpu.VMEM((vpad,), jnp.float32),
            pltpu.VMEM((num_fields, bpw), jnp.int32),
            pltpu.VMEM((ndense, bpw), jnp.float32),
            pltpu.VMEM((_L,), jnp.float32),
            pltpu.VMEM((bpw,), jnp.float32),
            pltpu.SemaphoreType.DMA,
        ],
        compiler_params=pltpu.CompilerParams(
            needs_layout_passes=False, use_tc_tiling_on_sc=True
        ),
    )
    def k(table_hbm, idx_hbm, dns_hbm, wb_hbm, out_hbm,
          table_v, idx_v, dns_v, wb_v, out_v, sem):
        wid = lax.axis_index("s") * 2 + lax.axis_index("c")
        base = wid * bpw
        c1 = pltpu.async_copy(table_hbm, table_v, sem)
        c2 = pltpu.async_copy(idx_hbm.at[:, pl.ds(base, bpw)], idx_v, sem)
        c3 = pltpu.async_copy(dns_hbm.at[:, pl.ds(base, bpw)], dns_v, sem)
        c4 = pltpu.async_copy(wb_hbm, wb_v, sem)
        c1.wait()
        c2.wait()
        c3.wait()
        c4.wait()

        lane = lax.iota(jnp.int32, _L)
        zero = jnp.zeros((_L,), jnp.int32)
        wvecs = [plsc.load_gather(wb_v, [zero + j]) for j in range(ndense)]
        bvec = plsc.load_gather(wb_v, [zero + ndense])

        def body(g, _):
            acc = bvec
            for j in range(ndense):
                acc = acc + dns_v[j, pl.ds(g * _L, _L)] * wvecs[j]
            for f in range(num_fields):
                ii = idx_v[f, pl.ds(g * _L, _L)]
                acc = acc + plsc.load_gather(table_v, [ii])
            out_v[pl.ds(g * _L, _L)] = _sigmoid(acc)
            return _

        lax.fori_loop(0, ngroups, body, None)
        pltpu.sync_copy(out_v, out_hbm.at[pl.ds(base, bpw)])

    return k(table, idx_t, dns_t, wb)


def kernel(dense_input, sparse_input, emb_table, fm_W, fm_b):
    B, ndense = dense_input.shape
    V = emb_table.shape[0]
    NW = 32

    idx_t = sparse_input.astype(jnp.int32).T
    dns_t = dense_input.astype(jnp.float32).T
    vpad = ((V + 127) // 128) * 128
    table = jnp.zeros((vpad,), jnp.float32).at[:V].set(emb_table[:, 0])
    wb = jnp.zeros((_L,), jnp.float32)
    wb = wb.at[:ndense].set(fm_W.reshape(ndense).astype(jnp.float32))
    wb = wb.at[ndense].set(fm_b.reshape(())[...].astype(jnp.float32))

    out = _run(table, idx_t, dns_t, wb, NW)
    return out.reshape(B, 1)
